# baseline (speedup 1.0000x reference)
import functools

import jax
import jax.numpy as jnp
from jax.experimental import pallas as pl
from jax.experimental.pallas import tpu as pltpu

_NSIGMA = 4.0


def _x_block_kernel(x_ref, o_ref):
    seq, bb, f = x_ref.shape
    n = jnp.float32(seq)
    ch = 16
    ni = seq // ch
    z = jnp.zeros((ch, bb, f), jnp.float32)

    def p1(i, acc):
        a1, a2 = acc
        c = x_ref[pl.ds(i * ch, ch)]
        return (a1 + c, a2 + c * c)

    a1, a2 = jax.lax.fori_loop(0, ni, p1, (z, z), unroll=2)
    s1 = jnp.sum(a1, axis=0, keepdims=True)
    s2 = jnp.sum(a2, axis=0, keepdims=True)
    mu = s1 / n
    std = jnp.sqrt(jnp.maximum(s2 / n - mu * mu, 0.0))
    cut = std * _NSIGMA
    lo = mu - cut
    hi = mu + cut

    def p2(i, acc):
        am, aq, ac = acc
        c = x_ref[pl.ds(i * ch, ch)]
        m = jnp.logical_or(c > hi, c < lo)
        sel = jnp.where(m, 0.0, c)
        one = jnp.where(m, 0.0, 1.0)
        return (am + sel, aq + sel * sel, ac + one)

    am, aq, ac = jax.lax.fori_loop(0, ni, p2, (z, z, z), unroll=2)
    sm = jnp.sum(am, axis=0, keepdims=True)
    sq = jnp.sum(aq, axis=0, keepdims=True)
    cnt = jnp.maximum(jnp.sum(ac, axis=0, keepdims=True), 1.0)
    mu2 = sm / cnt
    std2 = jnp.sqrt(jnp.maximum(sq / cnt - mu2 * mu2, 0.0))
    cut2 = std2 * _NSIGMA
    lo2 = mu2 - cut2
    hi2 = mu2 + cut2

    def p3(i, acc):
        a3, q3 = acc
        c = x_ref[pl.ds(i * ch, ch)]
        el = jnp.log1p(jnp.abs(c))
        x2 = jnp.minimum(jnp.maximum(lo2 - el, c), hi2 + el)
        o_ref[pl.ds(i * ch, ch)] = x2
        return (a3 + x2, q3 + x2 * x2)

    a3, q3 = jax.lax.fori_loop(0, ni, p3, (z, z), unroll=2)
    s3 = jnp.sum(a3, axis=0, keepdims=True)
    qq3 = jnp.sum(q3, axis=0, keepdims=True)
    m3 = s3 / n
    r3 = 1.0 / (jnp.sqrt(jnp.maximum(qq3 / n - m3 * m3, 0.0)) + 1e-6)
    o_ref[...] = (o_ref[...] - m3) * r3


def _y_kernel(bidx_ref, y_ref, yout_ref, lab_ref, *, num_classes):
    Y = y_ref[...]
    n = jnp.float32(Y.shape[0])
    s = jnp.sum(Y, axis=0)
    q = jnp.sum(Y * Y, axis=0)
    mu = s / n
    sd = jnp.sqrt(jnp.maximum(q / n - mu * mu, 0.0)) + 1e-6
    yout_ref[...] = (Y - mu[None, :]) / sd[None, :]
    lab = jnp.zeros(Y.shape, jnp.int32)
    for c in range(num_classes - 1):
        row = y_ref[pl.ds(bidx_ref[c], 1), :]
        lab = lab + (Y > row).astype(jnp.int32)
    lab_ref[...] = lab


def kernel(x, y, boundary_idx, batch_size):
    seq, b, f = x.shape
    bb = 8
    grid = b // bb
    x_out = pl.pallas_call(
        _x_block_kernel,
        grid=(grid,),
        in_specs=[pl.BlockSpec((seq, bb, f), lambda i: (0, i, 0))],
        out_specs=pl.BlockSpec((seq, bb, f), lambda i: (0, i, 0)),
        out_shape=jax.ShapeDtypeStruct((seq, b, f), jnp.float32),
        compiler_params=pltpu.CompilerParams(
            dimension_semantics=("parallel",),
        ),
    )(x)

    num_classes = boundary_idx.shape[0] + 1
    bidx = boundary_idx.astype(jnp.int32)
    y_out, labels = pl.pallas_call(
        functools.partial(_y_kernel, num_classes=num_classes),
        grid_spec=pltpu.PrefetchScalarGridSpec(
            num_scalar_prefetch=1,
            grid=(1,),
            in_specs=[pl.BlockSpec((seq, b), lambda i, bref: (0, 0))],
            out_specs=[
                pl.BlockSpec((seq, b), lambda i, bref: (0, 0)),
                pl.BlockSpec((seq, b), lambda i, bref: (0, 0)),
            ],
        ),
        out_shape=[
            jax.ShapeDtypeStruct((seq, b), jnp.float32),
            jax.ShapeDtypeStruct((seq, b), jnp.int32),
        ],
    )(bidx, y)
    return x_out, y_out, labels


# speedup vs baseline: 1.1764x; 1.1764x over previous
import functools

import jax
import jax.numpy as jnp
from jax.experimental import pallas as pl
from jax.experimental.pallas import tpu as pltpu

_NSIGMA = 4.0


def _x_block_kernel(x_ref, o_ref):
    seq, bb, f = x_ref.shape
    n = jnp.float32(seq)
    ch = 8
    ni = seq // ch
    z = jnp.zeros((ch, bb, f), jnp.float32)

    def p1(i, acc):
        a1, a2 = acc
        c = x_ref[pl.ds(i * ch, ch)]
        return (a1 + c, a2 + c * c)

    a1, a2 = jax.lax.fori_loop(0, ni, p1, (z, z), unroll=8)
    s1 = jnp.sum(a1, axis=0, keepdims=True)
    s2 = jnp.sum(a2, axis=0, keepdims=True)
    mu = s1 / n
    std = jnp.sqrt(jnp.maximum(s2 / n - mu * mu, 0.0))
    cut = std * _NSIGMA
    lo = mu - cut
    hi = mu + cut

    def p2(i, acc):
        am, aq, ac = acc
        c = x_ref[pl.ds(i * ch, ch)]
        m = jnp.logical_or(c > hi, c < lo)
        sel = jnp.where(m, 0.0, c)
        one = jnp.where(m, 0.0, 1.0)
        return (am + sel, aq + sel * sel, ac + one)

    am, aq, ac = jax.lax.fori_loop(0, ni, p2, (z, z, z), unroll=8)
    sm = jnp.sum(am, axis=0, keepdims=True)
    sq = jnp.sum(aq, axis=0, keepdims=True)
    cnt = jnp.maximum(jnp.sum(ac, axis=0, keepdims=True), 1.0)
    mu2 = sm / cnt
    std2 = jnp.sqrt(jnp.maximum(sq / cnt - mu2 * mu2, 0.0))
    cut2 = std2 * _NSIGMA
    lo2 = mu2 - cut2
    hi2 = mu2 + cut2

    def p3(i, acc):
        a3, q3 = acc
        c = x_ref[pl.ds(i * ch, ch)]
        el = jnp.log1p(jnp.abs(c))
        x2 = jnp.minimum(jnp.maximum(lo2 - el, c), hi2 + el)
        o_ref[pl.ds(i * ch, ch)] = x2
        return (a3 + x2, q3 + x2 * x2)

    a3, q3 = jax.lax.fori_loop(0, ni, p3, (z, z), unroll=8)
    s3 = jnp.sum(a3, axis=0, keepdims=True)
    qq3 = jnp.sum(q3, axis=0, keepdims=True)
    m3 = s3 / n
    r3 = 1.0 / (jnp.sqrt(jnp.maximum(qq3 / n - m3 * m3, 0.0)) + 1e-6)
    o_ref[...] = (o_ref[...] - m3) * r3


def _y_kernel(bidx_ref, y_ref, yout_ref, lab_ref, *, num_classes):
    Y = y_ref[...]
    n = jnp.float32(Y.shape[0])
    s = jnp.sum(Y, axis=0)
    q = jnp.sum(Y * Y, axis=0)
    mu = s / n
    sd = jnp.sqrt(jnp.maximum(q / n - mu * mu, 0.0)) + 1e-6
    yout_ref[...] = (Y - mu[None, :]) / sd[None, :]
    lab = jnp.zeros(Y.shape, jnp.int32)
    for c in range(num_classes - 1):
        row = y_ref[pl.ds(bidx_ref[c], 1), :]
        lab = lab + (Y > row).astype(jnp.int32)
    lab_ref[...] = lab


def kernel(x, y, boundary_idx, batch_size):
    seq, b, f = x.shape
    bb = 8
    grid = b // bb
    x_out = pl.pallas_call(
        _x_block_kernel,
        grid=(grid,),
        in_specs=[pl.BlockSpec((seq, bb, f), lambda i: (0, i, 0))],
        out_specs=pl.BlockSpec((seq, bb, f), lambda i: (0, i, 0)),
        out_shape=jax.ShapeDtypeStruct((seq, b, f), jnp.float32),
        compiler_params=pltpu.CompilerParams(
            dimension_semantics=("parallel",),
        ),
    )(x)

    num_classes = boundary_idx.shape[0] + 1
    bidx = boundary_idx.astype(jnp.int32)
    y_out, labels = pl.pallas_call(
        functools.partial(_y_kernel, num_classes=num_classes),
        grid_spec=pltpu.PrefetchScalarGridSpec(
            num_scalar_prefetch=1,
            grid=(1,),
            in_specs=[pl.BlockSpec((seq, b), lambda i, bref: (0, 0))],
            out_specs=[
                pl.BlockSpec((seq, b), lambda i, bref: (0, 0)),
                pl.BlockSpec((seq, b), lambda i, bref: (0, 0)),
            ],
        ),
        out_shape=[
            jax.ShapeDtypeStruct((seq, b), jnp.float32),
            jax.ShapeDtypeStruct((seq, b), jnp.int32),
        ],
    )(bidx, y)
    return x_out, y_out, labels


# speedup vs baseline: 1.2250x; 1.0414x over previous
import functools

import jax
import jax.numpy as jnp
from jax.experimental import pallas as pl
from jax.experimental.pallas import tpu as pltpu

_NSIGMA = 4.0


def _x_block_kernel(x_ref, o_ref):
    seq, bb, f = x_ref.shape
    n = jnp.float32(seq)
    ch = 8
    ni = seq // ch
    z = jnp.zeros((ch, bb, f), jnp.float32)

    def p1(i, acc):
        a1, a2 = acc
        c = x_ref[pl.ds(i * ch, ch)]
        return (a1 + c, a2 + c * c)

    a1, a2 = jax.lax.fori_loop(0, ni, p1, (z, z), unroll=16)
    s1 = jnp.sum(a1, axis=0, keepdims=True)
    s2 = jnp.sum(a2, axis=0, keepdims=True)
    mu = s1 / n
    std = jnp.sqrt(jnp.maximum(s2 / n - mu * mu, 0.0))
    cut = std * _NSIGMA
    lo = mu - cut
    hi = mu + cut

    def p2(i, acc):
        am, aq, ac = acc
        c = x_ref[pl.ds(i * ch, ch)]
        m = jnp.logical_or(c > hi, c < lo)
        sel = jnp.where(m, 0.0, c)
        one = jnp.where(m, 0.0, 1.0)
        return (am + sel, aq + sel * sel, ac + one)

    am, aq, ac = jax.lax.fori_loop(0, ni, p2, (z, z, z), unroll=16)
    sm = jnp.sum(am, axis=0, keepdims=True)
    sq = jnp.sum(aq, axis=0, keepdims=True)
    cnt = jnp.maximum(jnp.sum(ac, axis=0, keepdims=True), 1.0)
    mu2 = sm / cnt
    std2 = jnp.sqrt(jnp.maximum(sq / cnt - mu2 * mu2, 0.0))
    cut2 = std2 * _NSIGMA
    lo2 = mu2 - cut2
    hi2 = mu2 + cut2

    def p3(i, acc):
        a3, q3 = acc
        c = x_ref[pl.ds(i * ch, ch)]
        el = jnp.log1p(jnp.abs(c))
        x2 = jnp.minimum(jnp.maximum(lo2 - el, c), hi2 + el)
        o_ref[pl.ds(i * ch, ch)] = x2
        return (a3 + x2, q3 + x2 * x2)

    a3, q3 = jax.lax.fori_loop(0, ni, p3, (z, z), unroll=16)
    s3 = jnp.sum(a3, axis=0, keepdims=True)
    qq3 = jnp.sum(q3, axis=0, keepdims=True)
    m3 = s3 / n
    r3 = 1.0 / (jnp.sqrt(jnp.maximum(qq3 / n - m3 * m3, 0.0)) + 1e-6)
    o_ref[...] = (o_ref[...] - m3) * r3


def _y_kernel(bidx_ref, y_ref, yout_ref, lab_ref, *, num_classes):
    Y = y_ref[...]
    n = jnp.float32(Y.shape[0])
    s = jnp.sum(Y, axis=0)
    q = jnp.sum(Y * Y, axis=0)
    mu = s / n
    sd = jnp.sqrt(jnp.maximum(q / n - mu * mu, 0.0)) + 1e-6
    yout_ref[...] = (Y - mu[None, :]) / sd[None, :]
    lab = jnp.zeros(Y.shape, jnp.int32)
    for c in range(num_classes - 1):
        row = y_ref[pl.ds(bidx_ref[c], 1), :]
        lab = lab + (Y > row).astype(jnp.int32)
    lab_ref[...] = lab


def kernel(x, y, boundary_idx, batch_size):
    seq, b, f = x.shape
    bb = 8
    grid = b // bb
    x_out = pl.pallas_call(
        _x_block_kernel,
        grid=(grid,),
        in_specs=[pl.BlockSpec((seq, bb, f), lambda i: (0, i, 0))],
        out_specs=pl.BlockSpec((seq, bb, f), lambda i: (0, i, 0)),
        out_shape=jax.ShapeDtypeStruct((seq, b, f), jnp.float32),
        compiler_params=pltpu.CompilerParams(
            dimension_semantics=("parallel",),
        ),
    )(x)

    num_classes = boundary_idx.shape[0] + 1
    bidx = boundary_idx.astype(jnp.int32)
    y_out, labels = pl.pallas_call(
        functools.partial(_y_kernel, num_classes=num_classes),
        grid_spec=pltpu.PrefetchScalarGridSpec(
            num_scalar_prefetch=1,
            grid=(1,),
            in_specs=[pl.BlockSpec((seq, b), lambda i, bref: (0, 0))],
            out_specs=[
                pl.BlockSpec((seq, b), lambda i, bref: (0, 0)),
                pl.BlockSpec((seq, b), lambda i, bref: (0, 0)),
            ],
        ),
        out_shape=[
            jax.ShapeDtypeStruct((seq, b), jnp.float32),
            jax.ShapeDtypeStruct((seq, b), jnp.int32),
        ],
    )(bidx, y)
    return x_out, y_out, labels


# speedup vs baseline: 1.2517x; 1.0218x over previous
import functools

import jax
import jax.numpy as jnp
from jax.experimental import pallas as pl
from jax.experimental.pallas import tpu as pltpu

_NSIGMA = 4.0


def _x_block_kernel(x_ref, o_ref):
    seq, bb, f = x_ref.shape
    n = jnp.float32(seq)
    ch = 8
    ni = seq // ch
    z = jnp.zeros((ch, bb, f), jnp.float32)

    def p1(i, acc):
        a1, a2 = acc
        c = x_ref[pl.ds(i * ch, ch)]
        return (a1 + c, a2 + c * c)

    a1, a2 = jax.lax.fori_loop(0, ni, p1, (z, z), unroll=32)
    s1 = jnp.sum(a1, axis=0, keepdims=True)
    s2 = jnp.sum(a2, axis=0, keepdims=True)
    mu = s1 / n
    std = jnp.sqrt(jnp.maximum(s2 / n - mu * mu, 0.0))
    cut = std * _NSIGMA
    lo = mu - cut
    hi = mu + cut

    def p2(i, acc):
        am, aq, ac = acc
        c = x_ref[pl.ds(i * ch, ch)]
        m = jnp.logical_or(c > hi, c < lo)
        sel = jnp.where(m, 0.0, c)
        one = jnp.where(m, 0.0, 1.0)
        return (am + sel, aq + sel * sel, ac + one)

    am, aq, ac = jax.lax.fori_loop(0, ni, p2, (z, z, z), unroll=32)
    sm = jnp.sum(am, axis=0, keepdims=True)
    sq = jnp.sum(aq, axis=0, keepdims=True)
    cnt = jnp.maximum(jnp.sum(ac, axis=0, keepdims=True), 1.0)
    mu2 = sm / cnt
    std2 = jnp.sqrt(jnp.maximum(sq / cnt - mu2 * mu2, 0.0))
    cut2 = std2 * _NSIGMA
    lo2 = mu2 - cut2
    hi2 = mu2 + cut2

    def p3(i, acc):
        a3, q3 = acc
        c = x_ref[pl.ds(i * ch, ch)]
        el = jnp.log1p(jnp.abs(c))
        x2 = jnp.minimum(jnp.maximum(lo2 - el, c), hi2 + el)
        o_ref[pl.ds(i * ch, ch)] = x2
        return (a3 + x2, q3 + x2 * x2)

    a3, q3 = jax.lax.fori_loop(0, ni, p3, (z, z), unroll=32)
    s3 = jnp.sum(a3, axis=0, keepdims=True)
    qq3 = jnp.sum(q3, axis=0, keepdims=True)
    m3 = s3 / n
    r3 = 1.0 / (jnp.sqrt(jnp.maximum(qq3 / n - m3 * m3, 0.0)) + 1e-6)
    o_ref[...] = (o_ref[...] - m3) * r3


def _y_kernel(bidx_ref, y_ref, yout_ref, lab_ref, *, num_classes):
    Y = y_ref[...]
    n = jnp.float32(Y.shape[0])
    s = jnp.sum(Y, axis=0)
    q = jnp.sum(Y * Y, axis=0)
    mu = s / n
    sd = jnp.sqrt(jnp.maximum(q / n - mu * mu, 0.0)) + 1e-6
    yout_ref[...] = (Y - mu[None, :]) / sd[None, :]
    lab = jnp.zeros(Y.shape, jnp.int32)
    for c in range(num_classes - 1):
        row = y_ref[pl.ds(bidx_ref[c], 1), :]
        lab = lab + (Y > row).astype(jnp.int32)
    lab_ref[...] = lab


def kernel(x, y, boundary_idx, batch_size):
    seq, b, f = x.shape
    bb = 8
    grid = b // bb
    x_out = pl.pallas_call(
        _x_block_kernel,
        grid=(grid,),
        in_specs=[pl.BlockSpec((seq, bb, f), lambda i: (0, i, 0))],
        out_specs=pl.BlockSpec((seq, bb, f), lambda i: (0, i, 0)),
        out_shape=jax.ShapeDtypeStruct((seq, b, f), jnp.float32),
        compiler_params=pltpu.CompilerParams(
            dimension_semantics=("parallel",),
        ),
    )(x)

    num_classes = boundary_idx.shape[0] + 1
    bidx = boundary_idx.astype(jnp.int32)
    y_out, labels = pl.pallas_call(
        functools.partial(_y_kernel, num_classes=num_classes),
        grid_spec=pltpu.PrefetchScalarGridSpec(
            num_scalar_prefetch=1,
            grid=(1,),
            in_specs=[pl.BlockSpec((seq, b), lambda i, bref: (0, 0))],
            out_specs=[
                pl.BlockSpec((seq, b), lambda i, bref: (0, 0)),
                pl.BlockSpec((seq, b), lambda i, bref: (0, 0)),
            ],
        ),
        out_shape=[
            jax.ShapeDtypeStruct((seq, b), jnp.float32),
            jax.ShapeDtypeStruct((seq, b), jnp.int32),
        ],
    )(bidx, y)
    return x_out, y_out, labels


# speedup vs baseline: 1.2635x; 1.0094x over previous
import functools

import jax
import jax.numpy as jnp
from jax.experimental import pallas as pl
from jax.experimental.pallas import tpu as pltpu

_NSIGMA = 4.0


def _x_block_kernel(x_ref, o_ref):
    seq, bb, f = x_ref.shape
    n = jnp.float32(seq)
    ch = 8
    ni = seq // ch
    z = jnp.zeros((ch, bb, f), jnp.float32)

    def p1(i, acc):
        a1, a2 = acc
        c = x_ref[pl.ds(i * ch, ch)]
        return (a1 + c, a2 + c * c)

    a1, a2 = jax.lax.fori_loop(0, ni, p1, (z, z), unroll=64)
    s1 = jnp.sum(a1, axis=0, keepdims=True)
    s2 = jnp.sum(a2, axis=0, keepdims=True)
    mu = s1 / n
    std = jnp.sqrt(jnp.maximum(s2 / n - mu * mu, 0.0))
    cut = std * _NSIGMA
    lo = mu - cut
    hi = mu + cut

    def p2(i, acc):
        am, aq, ac = acc
        c = x_ref[pl.ds(i * ch, ch)]
        m = jnp.logical_or(c > hi, c < lo)
        sel = jnp.where(m, 0.0, c)
        one = jnp.where(m, 0.0, 1.0)
        return (am + sel, aq + sel * sel, ac + one)

    am, aq, ac = jax.lax.fori_loop(0, ni, p2, (z, z, z), unroll=64)
    sm = jnp.sum(am, axis=0, keepdims=True)
    sq = jnp.sum(aq, axis=0, keepdims=True)
    cnt = jnp.maximum(jnp.sum(ac, axis=0, keepdims=True), 1.0)
    mu2 = sm / cnt
    std2 = jnp.sqrt(jnp.maximum(sq / cnt - mu2 * mu2, 0.0))
    cut2 = std2 * _NSIGMA
    lo2 = mu2 - cut2
    hi2 = mu2 + cut2

    def p3(i, acc):
        a3, q3 = acc
        c = x_ref[pl.ds(i * ch, ch)]
        el = jnp.log1p(jnp.abs(c))
        x2 = jnp.minimum(jnp.maximum(lo2 - el, c), hi2 + el)
        o_ref[pl.ds(i * ch, ch)] = x2
        return (a3 + x2, q3 + x2 * x2)

    a3, q3 = jax.lax.fori_loop(0, ni, p3, (z, z), unroll=64)
    s3 = jnp.sum(a3, axis=0, keepdims=True)
    qq3 = jnp.sum(q3, axis=0, keepdims=True)
    m3 = s3 / n
    r3 = 1.0 / (jnp.sqrt(jnp.maximum(qq3 / n - m3 * m3, 0.0)) + 1e-6)
    o_ref[...] = (o_ref[...] - m3) * r3


def _y_kernel(bidx_ref, y_ref, yout_ref, lab_ref, *, num_classes):
    Y = y_ref[...]
    n = jnp.float32(Y.shape[0])
    s = jnp.sum(Y, axis=0)
    q = jnp.sum(Y * Y, axis=0)
    mu = s / n
    sd = jnp.sqrt(jnp.maximum(q / n - mu * mu, 0.0)) + 1e-6
    yout_ref[...] = (Y - mu[None, :]) / sd[None, :]
    lab = jnp.zeros(Y.shape, jnp.int32)
    for c in range(num_classes - 1):
        row = y_ref[pl.ds(bidx_ref[c], 1), :]
        lab = lab + (Y > row).astype(jnp.int32)
    lab_ref[...] = lab


def kernel(x, y, boundary_idx, batch_size):
    seq, b, f = x.shape
    bb = 8
    grid = b // bb
    x_out = pl.pallas_call(
        _x_block_kernel,
        grid=(grid,),
        in_specs=[pl.BlockSpec((seq, bb, f), lambda i: (0, i, 0))],
        out_specs=pl.BlockSpec((seq, bb, f), lambda i: (0, i, 0)),
        out_shape=jax.ShapeDtypeStruct((seq, b, f), jnp.float32),
        compiler_params=pltpu.CompilerParams(
            dimension_semantics=("parallel",),
        ),
    )(x)

    num_classes = boundary_idx.shape[0] + 1
    bidx = boundary_idx.astype(jnp.int32)
    y_out, labels = pl.pallas_call(
        functools.partial(_y_kernel, num_classes=num_classes),
        grid_spec=pltpu.PrefetchScalarGridSpec(
            num_scalar_prefetch=1,
            grid=(1,),
            in_specs=[pl.BlockSpec((seq, b), lambda i, bref: (0, 0))],
            out_specs=[
                pl.BlockSpec((seq, b), lambda i, bref: (0, 0)),
                pl.BlockSpec((seq, b), lambda i, bref: (0, 0)),
            ],
        ),
        out_shape=[
            jax.ShapeDtypeStruct((seq, b), jnp.float32),
            jax.ShapeDtypeStruct((seq, b), jnp.int32),
        ],
    )(bidx, y)
    return x_out, y_out, labels
